# baseline (speedup 1.0000x reference)
import functools

import jax
import jax.numpy as jnp
from jax import lax
from jax.experimental import pallas as pl
from jax.experimental.pallas import tpu as pltpu, tpu_sc as plsc

_B = 16
_NC = 6
_CW = 36864
_NISS = 8
_IMG_CHUNKS_PER_CORE = 576
_MSK_CHUNKS_PER_CORE = 192
_CHUNKS_PER_CORE = _IMG_CHUNKS_PER_CORE + _MSK_CHUNKS_PER_CORE
_PER_ISS = _CHUNKS_PER_CORE // _NISS
_NG = _PER_ISS // 2


def _sc_body(cam_hbm, imgs_hbm, masks_hbm, imgs_out, masks_out,
             cam_v, zstage, slots, zeros, sem_cam, sem_z, sem_in, sem_out):
    cid = lax.axis_index("c")
    sid = lax.axis_index("s")

    pltpu.make_async_copy(cam_hbm, cam_v.at[pl.ds(0, 16)], sem_cam).start()

    @pl.when(sid == 0)
    def _init_zeros():
        def _zf(i, _):
            zstage[pl.ds(i * 16, 16)] = jnp.zeros((16,), jnp.float32)
            return 0
        lax.fori_loop(0, _CW // 16, _zf, 0)
        pltpu.make_async_copy(zstage, zeros, sem_z).start()
        pltpu.make_async_copy(zstage, zeros, sem_z).wait()

    pltpu.make_async_copy(cam_hbm, cam_v.at[pl.ds(0, 16)], sem_cam).wait()
    plsc.subcore_barrier()

    @pl.when(sid < _NISS)
    def _issue():
        def chunk_info(j):
            ch = j * _NISS + sid
            is_img = ch < _IMG_CHUNKS_PER_CORE
            iidx = cid * _IMG_CHUNKS_PER_CORE + ch
            midx = cid * _MSK_CHUNKS_PER_CORE + (ch - _IMG_CHUNKS_PER_CORE)
            row = jnp.where(is_img, iidx // 12, midx // 4)
            keep = cam_v[pl.ds(row // _NC, 16)][0] != row % _NC
            return is_img, iidx, midx, keep

        def start_in(j, slot_i):
            is_img, iidx, midx, keep = chunk_info(j)
            slot = slots.at[sid * 2 + slot_i]
            pl.when(keep & is_img)(
                pltpu.make_async_copy(imgs_hbm.at[iidx], slot, sem_in).start)
            pl.when(keep & jnp.logical_not(is_img))(
                pltpu.make_async_copy(masks_hbm.at[midx], slot, sem_in).start)

        def wait_in_start_out(j, slot_i):
            is_img, iidx, midx, keep = chunk_info(j)
            slot = slots.at[sid * 2 + slot_i]
            pl.when(keep)(
                pltpu.make_async_copy(imgs_hbm.at[0], slot, sem_in).wait)
            pl.when(is_img & keep)(
                pltpu.make_async_copy(slot, imgs_out.at[iidx], sem_out).start)
            pl.when(is_img & jnp.logical_not(keep))(
                pltpu.make_async_copy(zeros, imgs_out.at[iidx], sem_out).start)
            pl.when(jnp.logical_not(is_img) & keep)(
                pltpu.make_async_copy(slot, masks_out.at[midx], sem_out).start)
            pl.when(jnp.logical_not(is_img) & jnp.logical_not(keep))(
                pltpu.make_async_copy(zeros, masks_out.at[midx], sem_out).start)

        def wait_out():
            pltpu.make_async_copy(zeros, imgs_out.at[0], sem_out).wait()

        start_in(0, 0)
        start_in(1, 1)

        def _group(g, _):
            j0 = 2 * g
            wait_in_start_out(j0, 0)

            @pl.when(g > 0)
            def _w_even():
                wait_out()
                start_in(j0 + 1, 1)

            wait_in_start_out(j0 + 1, 1)
            wait_out()

            @pl.when(g < _NG - 1)
            def _w_odd():
                start_in(j0 + 2, 0)
            return 0

        lax.fori_loop(0, _NG, _group, 0)
        wait_out()


def kernel(imgs, grids, masks, cam_choice):
    B, NC, C, H, W = imgs.shape
    imgs3 = imgs.reshape(B * NC * 12, _CW)
    masks2 = masks.reshape(B * NC * 4, _CW)

    mesh = plsc.VectorSubcoreMesh(core_axis_name="c", subcore_axis_name="s")
    sc = functools.partial(
        pl.kernel,
        out_type=[
            jax.ShapeDtypeStruct(imgs3.shape, imgs3.dtype),
            jax.ShapeDtypeStruct(masks2.shape, masks2.dtype),
        ],
        mesh=mesh,
        scratch_types=[
            pltpu.VMEM((32,), jnp.int32),
            pltpu.VMEM((_CW,), jnp.float32),
            pltpu.MemorySpace.VMEM_SHARED((_NISS * 2, _CW), jnp.float32),
            pltpu.MemorySpace.VMEM_SHARED((_CW,), jnp.float32),
            pltpu.SemaphoreType.DMA,
            pltpu.SemaphoreType.DMA,
            pltpu.SemaphoreType.DMA,
            pltpu.SemaphoreType.DMA,
        ],
    )(_sc_body)

    imgs_out, masks_out = sc(cam_choice.astype(jnp.int32), imgs3, masks2)
    return (imgs_out.reshape(imgs.shape), grids, masks_out.reshape(masks.shape))


# speedup vs baseline: 1.0439x; 1.0439x over previous
import functools

import jax
import jax.numpy as jnp
from jax import lax
from jax.experimental import pallas as pl
from jax.experimental.pallas import tpu as pltpu

_B = 16
_NC = 6
_W = 36864
_S = 48
_GI = 24
_GM = 8
_NBUF = 3


def _body(cam_ref, imgs_ref, masks_ref, imgs_out, masks_out,
          b0, b1, b2, sem_in, sem_out):
    bufs = (b0, b1, b2)

    def src(g):
        if g < _GI:
            return imgs_ref.at[:, g, :], imgs_out.at[:, g, :]
        m = g - _GI
        return masks_ref.at[:, m, :], masks_out.at[:, m, :]

    def rows_per_seg(g):
        return None

    def start_in(g):
        s, _ = src(g)
        pltpu.make_async_copy(s, bufs[g % _NBUF], sem_in).start()

    def finish(g):
        s, d = src(g)
        buf = bufs[g % _NBUF]
        pltpu.make_async_copy(s, buf, sem_in).wait()
        for srow in range(_S):
            if g < _GI:
                row = srow * _GI + g
                sl = row // 12
            else:
                row = srow * _GM + (g - _GI)
                sl = row // 4
            keep = cam_ref[sl // _NC] != (sl % _NC)

            @pl.when(jnp.logical_not(keep))
            def _z():
                buf[srow, :] = jnp.zeros((_W,), jnp.float32)

        pltpu.make_async_copy(buf, d, sem_out).start()

    def wait_out(g):
        _, d = src(g)
        pltpu.make_async_copy(bufs[g % _NBUF], d, sem_out).wait()

    n = _GI + _GM
    for g in range(min(_NBUF, n)):
        start_in(g)
    waited = set()
    for g in range(n):
        if g >= 1 and (g + _NBUF - 1) < n:
            wait_out(g - 1)
            waited.add(g - 1)
            start_in(g + _NBUF - 1)
        finish(g)
    for g in range(n):
        if g not in waited:
            wait_out(g)


def kernel(imgs, grids, masks, cam_choice):
    B, NC, C, H, W = imgs.shape
    imgs4 = imgs.reshape(_S, _GI, _W)
    masks3 = masks.reshape(_S, _GM, _W)

    imgs_out, masks_out = pl.pallas_call(
        _body,
        grid_spec=pltpu.PrefetchScalarGridSpec(
            num_scalar_prefetch=1,
            grid=(1,),
            in_specs=[
                pl.BlockSpec(memory_space=pl.ANY),
                pl.BlockSpec(memory_space=pl.ANY),
            ],
            out_specs=[
                pl.BlockSpec(memory_space=pl.ANY),
                pl.BlockSpec(memory_space=pl.ANY),
            ],
            scratch_shapes=[
                pltpu.VMEM((_S, _W), jnp.float32),
                pltpu.VMEM((_S, _W), jnp.float32),
                pltpu.VMEM((_S, _W), jnp.float32),
                pltpu.SemaphoreType.DMA,
                pltpu.SemaphoreType.DMA,
            ],
        ),
        out_shape=[
            jax.ShapeDtypeStruct(imgs4.shape, imgs4.dtype),
            jax.ShapeDtypeStruct(masks3.shape, masks3.dtype),
        ],
    )(cam_choice.astype(jnp.int32), imgs4, masks3)

    return (imgs_out.reshape(imgs.shape), grids, masks_out.reshape(masks.shape))


# speedup vs baseline: 1.2013x; 1.1508x over previous
import functools

import jax
import jax.numpy as jnp
from jax import lax
from jax.experimental import pallas as pl
from jax.experimental.pallas import tpu as pltpu, tpu_sc as plsc

_B = 16
_NC = 6
_CW = 73728
_NISS = 8
_IMG_CHUNKS_PER_CORE = 288
_NGROUPS = _IMG_CHUNKS_PER_CORE // _NISS


def _sc_body(cam_hbm, imgs_hbm, imgs_out,
             cam_v, zstage, slots, zeros, sem_cam, sem_z, sem_in, sem_out):
    cid = lax.axis_index("c")
    sid = lax.axis_index("s")

    pltpu.make_async_copy(cam_hbm, cam_v.at[pl.ds(0, 16)], sem_cam).start()

    @pl.when(sid == 0)
    def _init_zeros():
        def _zf(i, _):
            zstage[pl.ds(i * 16, 16)] = jnp.zeros((16,), jnp.float32)
            return 0
        lax.fori_loop(0, _CW // 16, _zf, 0)
        pltpu.make_async_copy(zstage, zeros, sem_z).start()
        pltpu.make_async_copy(zstage, zeros, sem_z).wait()

    pltpu.make_async_copy(cam_hbm, cam_v.at[pl.ds(0, 16)], sem_cam).wait()
    plsc.subcore_barrier()

    @pl.when(sid < _NISS)
    def _issue():
        slot = slots.at[sid]

        def _group(g, _):
            ch = g * _NISS + sid
            idx = cid * _IMG_CHUNKS_PER_CORE + ch
            row = idx // 6
            keep = cam_v[pl.ds(row // _NC, 16)][0] != row % _NC

            @pl.when(keep)
            def _copy():
                pltpu.make_async_copy(imgs_hbm.at[idx], slot, sem_in).start()
                pltpu.make_async_copy(imgs_hbm.at[idx], slot, sem_in).wait()
                pltpu.make_async_copy(slot, imgs_out.at[idx], sem_out).start()
                pltpu.make_async_copy(slot, imgs_out.at[idx], sem_out).wait()

            @pl.when(jnp.logical_not(keep))
            def _blank():
                pltpu.make_async_copy(zeros, imgs_out.at[idx], sem_out).start()
                pltpu.make_async_copy(zeros, imgs_out.at[idx], sem_out).wait()

            return 0

        lax.fori_loop(0, _NGROUPS, _group, 0)


def _tc_body(cam_ref, masks_ref, masks_out_ref):
    p = pl.program_id(0)
    keep = jnp.where(cam_ref[p // _NC] == p % _NC, 0.0, 1.0).astype(jnp.float32)
    masks_out_ref[...] = masks_ref[...] * keep


def kernel(imgs, grids, masks, cam_choice):
    B, NC, C, H, W = imgs.shape
    cam32 = cam_choice.astype(jnp.int32)
    imgs3 = imgs.reshape(B * NC * 6, _CW)
    masks2 = masks.reshape(B * NC, 128, 1152)

    mesh = plsc.VectorSubcoreMesh(core_axis_name="c", subcore_axis_name="s")
    sc = functools.partial(
        pl.kernel,
        out_type=jax.ShapeDtypeStruct(imgs3.shape, imgs3.dtype),
        mesh=mesh,
        scratch_types=[
            pltpu.VMEM((32,), jnp.int32),
            pltpu.VMEM((_CW,), jnp.float32),
            pltpu.MemorySpace.VMEM_SHARED((_NISS, _CW), jnp.float32),
            pltpu.MemorySpace.VMEM_SHARED((_CW,), jnp.float32),
            pltpu.SemaphoreType.DMA,
            pltpu.SemaphoreType.DMA,
            pltpu.SemaphoreType.DMA,
            pltpu.SemaphoreType.DMA,
        ],
    )(_sc_body)

    imgs_out = sc(cam32, imgs3)

    masks_out = pl.pallas_call(
        _tc_body,
        grid_spec=pltpu.PrefetchScalarGridSpec(
            num_scalar_prefetch=1,
            grid=(B * NC,),
            in_specs=[pl.BlockSpec((1, 128, 1152), lambda p, cam: (p, 0, 0))],
            out_specs=pl.BlockSpec((1, 128, 1152), lambda p, cam: (p, 0, 0)),
        ),
        out_shape=jax.ShapeDtypeStruct(masks2.shape, masks2.dtype),
    )(cam32, masks2)

    return (imgs_out.reshape(imgs.shape), grids, masks_out.reshape(masks.shape))


# speedup vs baseline: 1.2508x; 1.0412x over previous
import functools

import jax
import jax.numpy as jnp
from jax import lax
from jax.experimental import pallas as pl
from jax.experimental.pallas import tpu as pltpu, tpu_sc as plsc

_B = 16
_NC = 6
_CW = 110592
_ZW = 36864
_NISS = 6
_IMG_CHUNKS_PER_CORE = 192
_NGROUPS = _IMG_CHUNKS_PER_CORE // _NISS


def _sc_body(cam_hbm, imgs_hbm, imgs_out,
             cam_v, zstage, slots, zeros, sem_cam, sem_z, sem_in, sem_out):
    cid = lax.axis_index("c")
    sid = lax.axis_index("s")

    pltpu.make_async_copy(cam_hbm, cam_v.at[pl.ds(0, 16)], sem_cam).start()

    @pl.when(sid == 0)
    def _init_zeros():
        def _zf(i, _):
            zstage[pl.ds(i * 16, 16)] = jnp.zeros((16,), jnp.float32)
            return 0
        lax.fori_loop(0, _ZW // 16, _zf, 0)
        pltpu.make_async_copy(zstage, zeros, sem_z).start()
        pltpu.make_async_copy(zstage, zeros, sem_z).wait()

    pltpu.make_async_copy(cam_hbm, cam_v.at[pl.ds(0, 16)], sem_cam).wait()
    plsc.subcore_barrier()

    @pl.when(sid < _NISS)
    def _issue():
        slot = slots.at[sid]

        def _group(g, _):
            ch = g * _NISS + sid
            idx = cid * _IMG_CHUNKS_PER_CORE + ch
            row = idx // 4
            keep = cam_v[pl.ds(row // _NC, 16)][0] != row % _NC

            @pl.when(keep)
            def _copy():
                pltpu.make_async_copy(imgs_hbm.at[idx], slot, sem_in).start()
                pltpu.make_async_copy(imgs_hbm.at[idx], slot, sem_in).wait()
                pltpu.make_async_copy(slot, imgs_out.at[idx], sem_out).start()
                pltpu.make_async_copy(slot, imgs_out.at[idx], sem_out).wait()

            @pl.when(jnp.logical_not(keep))
            def _blank():
                for z in range(3):
                    pltpu.make_async_copy(
                        zeros, imgs_out.at[idx, pl.ds(z * _ZW, _ZW)],
                        sem_out).start()
                for z in range(3):
                    pltpu.make_async_copy(
                        zeros, imgs_out.at[idx, pl.ds(z * _ZW, _ZW)],
                        sem_out).wait()

            return 0

        lax.fori_loop(0, _NGROUPS, _group, 0)


def _tc_body(cam_ref, masks_ref, masks_out_ref):
    p = pl.program_id(0)
    keep = jnp.where(cam_ref[p // _NC] == p % _NC, 0.0, 1.0).astype(jnp.float32)
    masks_out_ref[...] = masks_ref[...] * keep


def kernel(imgs, grids, masks, cam_choice):
    B, NC, C, H, W = imgs.shape
    cam32 = cam_choice.astype(jnp.int32)
    imgs3 = imgs.reshape(B * NC * 4, _CW)
    masks2 = masks.reshape(B * NC, 128, 1152)

    mesh = plsc.VectorSubcoreMesh(core_axis_name="c", subcore_axis_name="s")
    sc = functools.partial(
        pl.kernel,
        out_type=jax.ShapeDtypeStruct(imgs3.shape, imgs3.dtype),
        mesh=mesh,
        scratch_types=[
            pltpu.VMEM((32,), jnp.int32),
            pltpu.VMEM((_ZW,), jnp.float32),
            pltpu.MemorySpace.VMEM_SHARED((_NISS, _CW), jnp.float32),
            pltpu.MemorySpace.VMEM_SHARED((_ZW,), jnp.float32),
            pltpu.SemaphoreType.DMA,
            pltpu.SemaphoreType.DMA,
            pltpu.SemaphoreType.DMA,
            pltpu.SemaphoreType.DMA,
        ],
    )(_sc_body)

    imgs_out = sc(cam32, imgs3)

    masks_out = pl.pallas_call(
        _tc_body,
        grid_spec=pltpu.PrefetchScalarGridSpec(
            num_scalar_prefetch=1,
            grid=(B * NC,),
            in_specs=[pl.BlockSpec((1, 128, 1152), lambda p, cam: (p, 0, 0))],
            out_specs=pl.BlockSpec((1, 128, 1152), lambda p, cam: (p, 0, 0)),
        ),
        out_shape=jax.ShapeDtypeStruct(masks2.shape, masks2.dtype),
    )(cam32, masks2)

    return (imgs_out.reshape(imgs.shape), grids, masks_out.reshape(masks.shape))
